# baseline (speedup 1.0000x reference)
import jax
import jax.numpy as jnp
from jax import lax
from jax.experimental import pallas as pl
from jax.experimental.pallas import tpu as pltpu
from jax.experimental.pallas import tpu_sc as plsc

_B, _S = 4096, 200
_N = _B * _S
_D = 16
_K = 5
_NC, _NS = 2, 16
_NW = _NC * _NS
_RT = _S // 8
_Q1 = 100001
_HU = 4
_HP = _HU * 128

_BBLK = 2048


def _btab_body(bb_ref, bg_ref, out_ref):
    b0 = bb_ref[...]
    g = jax.nn.softplus(bg_ref[...])
    b1 = b0 + g[0:1, :]
    b2 = b1 + g[1:2, :]
    b3 = b2 + g[2:3, :]
    z = jnp.zeros_like(b0)
    out_ref[...] = jnp.concatenate([b0, b1, b2, b3, z, z, z, z], axis=0)


def _make_btab(beta_base, beta_gaps):
    grid = (_Q1 + _BBLK - 1) // _BBLK
    btT = pl.pallas_call(
        _btab_body,
        grid=(grid,),
        in_specs=[
            pl.BlockSpec((1, _BBLK), lambda i: (0, i)),
            pl.BlockSpec((3, _BBLK), lambda i: (0, i)),
        ],
        out_specs=pl.BlockSpec((8, _BBLK), lambda i: (0, i)),
        out_shape=jax.ShapeDtypeStruct((8, _Q1), jnp.float32),
    )(beta_base.T, beta_gaps.T)
    return btT


def _sc_body(sid_hbm, qid_hbm, th_hbm, q_hbm,
             t5, a5, b4, l5, p5,
             sidx0, sidx1, qidx0, qidx1,
             thv0, thv1, qv0, qv1,
             thT0, thT1, alT0, alT1, btT0, btT1,
             lgT0, lgT1, prT0, prT1,
             semg0, semg1, semo0, semo1):
    w = lax.axis_index("s") * _NC + lax.axis_index("c")
    lane = jnp.arange(16, dtype=jnp.int32)
    cols = [jnp.full((16,), d, jnp.int32) for d in range(20)]
    SIDX = (sidx0, sidx1)
    QIDX = (qidx0, qidx1)
    THV = (thv0, thv1)
    QV = (qv0, qv1)
    THT = (thT0, thT1)
    ALT = (alT0, alT1)
    BTT = (btT0, btT1)
    LGT = (lgT0, lgT1)
    PRT = (prT0, prT1)
    SEMG = (semg0, semg1)
    SEMO = (semo0, semo1)

    def fire(rt, h, par):
        ibase = (rt * _NW + w) * 8 + h * _HU
        pltpu.sync_copy(sid_hbm.at[pl.ds(ibase, _HU)], SIDX[par])
        pltpu.sync_copy(qid_hbm.at[pl.ds(ibase, _HU)], QIDX[par])
        for u in range(_HU):
            dst = pl.ds(u * 128, 128)
            pltpu.async_copy(th_hbm.at[SIDX[par].at[u]], THV[par].at[dst], SEMG[par])
            pltpu.async_copy(q_hbm.at[QIDX[par].at[u]], QV[par].at[dst], SEMG[par])

    def drain_gathers(par):
        pltpu.make_async_copy(th_hbm.at[pl.ds(0, _HP)], THV[par], SEMG[par]).wait()
        pltpu.make_async_copy(q_hbm.at[pl.ds(0, _HP)], QV[par], SEMG[par]).wait()

    def drain_out(par):
        for _i in range(4 * _HU):
            pltpu.make_async_copy(t5.at[0, 0, 0], THT[par].at[0, pl.ds(0, 8)],
                                  SEMO[par]).wait()
        for _i in range(_HU):
            pltpu.make_async_copy(b4.at[0, 0], BTT[par].at[0], SEMO[par]).wait()
        for _i in range(2 * _K):
            pltpu.make_async_copy(b4.at[0, 0], LGT[par].at[0], SEMO[par]).wait()

    def compute_half(rt, h, par):
        thv, qv = THV[par], QV[par]
        thT, alT, btT = THT[par], ALT[par], BTT[par]
        lgT, prT = LGT[par], PRT[par]
        sem_o = SEMO[par]

        @plsc.parallel_loop(0, _HU * 8, unroll=2)
        def _body(i):
            u = i // 8
            rows = i * 16 + lane
            sl = pl.ds((i % 8) * 16, 16)
            prods = []
            for d in range(16):
                tv = plsc.load_gather(thv, [rows, cols[d]])
                av = jnp.exp(0.3 * plsc.load_gather(qv, [rows, cols[d]]))
                thT[u, d, sl] = tv
                alT[u, d, sl] = av
                prods.append(av * tv)
            while len(prods) > 1:
                prods = [a + b for a, b in zip(prods[::2], prods[1::2])]
            at = prods[0]
            bv = []
            for k in range(4):
                b = plsc.load_gather(qv, [rows, cols[16 + k]])
                btT[u, k, sl] = b
                bv.append(b)
            c1 = at - bv[0]
            c2 = c1 + at - bv[1]
            c3 = c2 + at - bv[2]
            c4 = c3 + at - bv[3]
            z = jnp.zeros((16,), jnp.float32)
            m = jnp.maximum(jnp.maximum(jnp.maximum(jnp.maximum(z, c1), c2), c3), c4)
            e0 = jnp.exp(z - m)
            e1 = jnp.exp(c1 - m)
            e2 = jnp.exp(c2 - m)
            e3 = jnp.exp(c3 - m)
            e4 = jnp.exp(c4 - m)
            r = 1.0 / (e0 + e1 + e2 + e3 + e4)
            lgT[0, u, sl] = z
            lgT[1, u, sl] = c1
            lgT[2, u, sl] = c2
            lgT[3, u, sl] = c3
            lgT[4, u, sl] = c4
            prT[0, u, sl] = e0 * r
            prT[1, u, sl] = e1 * r
            prT[2, u, sl] = e2 * r
            prT[3, u, sl] = e3 * r
            prT[4, u, sl] = e4 * r

        for u in range(_HU):
            s = rt * 8 + h * _HU + u
            pltpu.async_copy(thT.at[u, pl.ds(0, 8)], t5.at[s, 0, w], sem_o)
            pltpu.async_copy(thT.at[u, pl.ds(8, 8)], t5.at[s, 1, w], sem_o)
            pltpu.async_copy(alT.at[u, pl.ds(0, 8)], a5.at[s, 0, w], sem_o)
            pltpu.async_copy(alT.at[u, pl.ds(8, 8)], a5.at[s, 1, w], sem_o)
            pltpu.async_copy(btT.at[u], b4.at[s, w], sem_o)
        osl = pl.ds(h * _HU, _HU)
        for k in range(_K):
            pltpu.async_copy(lgT.at[k], l5.at[k, rt, w, osl], sem_o)
            pltpu.async_copy(prT.at[k], p5.at[k, rt, w, osl], sem_o)

    fire(0, 0, 0)

    def step(rt, _):
        fire(rt, 1, 1)
        drain_gathers(0)

        @pl.when(rt > 0)
        def _():
            drain_out(0)

        compute_half(rt, 0, 0)
        fire(jnp.minimum(rt + 1, _RT - 1), 0, 0)
        drain_gathers(1)

        @pl.when(rt > 0)
        def _():
            drain_out(1)

        compute_half(rt, 1, 1)
        return 0

    lax.fori_loop(0, _RT, step, 0)
    drain_gathers(0)
    drain_out(0)
    drain_out(1)


def _sc_run(sid_sc, qid_sc, theta_table, qtab):
    mesh = plsc.VectorSubcoreMesh(core_axis_name="c", subcore_axis_name="s")
    f32 = jnp.float32
    return pl.kernel(
        _sc_body,
        out_type=(
            jax.ShapeDtypeStruct((_S, 2, 32, 8, 128), f32),
            jax.ShapeDtypeStruct((_S, 2, 32, 8, 128), f32),
            jax.ShapeDtypeStruct((_S, 32, 4, 128), f32),
            jax.ShapeDtypeStruct((_K, _RT, 32, 8, 128), f32),
            jax.ShapeDtypeStruct((_K, _RT, 32, 8, 128), f32),
        ),
        mesh=mesh,
        scratch_types=[
            pltpu.VMEM((_HU, 128), jnp.int32),
            pltpu.VMEM((_HU, 128), jnp.int32),
            pltpu.VMEM((_HU, 128), jnp.int32),
            pltpu.VMEM((_HU, 128), jnp.int32),
            pltpu.VMEM((_HP, 16), f32),
            pltpu.VMEM((_HP, 16), f32),
            pltpu.VMEM((_HP, 24), f32),
            pltpu.VMEM((_HP, 24), f32),
            pltpu.VMEM((_HU, 16, 128), f32),
            pltpu.VMEM((_HU, 16, 128), f32),
            pltpu.VMEM((_HU, 16, 128), f32),
            pltpu.VMEM((_HU, 16, 128), f32),
            pltpu.VMEM((_HU, 4, 128), f32),
            pltpu.VMEM((_HU, 4, 128), f32),
            pltpu.VMEM((_K, _HU, 128), f32),
            pltpu.VMEM((_K, _HU, 128), f32),
            pltpu.VMEM((_K, _HU, 128), f32),
            pltpu.VMEM((_K, _HU, 128), f32),
            pltpu.SemaphoreType.DMA,
            pltpu.SemaphoreType.DMA,
            pltpu.SemaphoreType.DMA,
            pltpu.SemaphoreType.DMA,
        ],
        compiler_params=pltpu.CompilerParams(
            use_tc_tiling_on_sc=False, needs_layout_passes=False),
    )(sid_sc, qid_sc, theta_table, qtab)


@jax.jit
def kernel(student_ids, questions, responses, theta_table, alpha_raw, beta_base, beta_gaps):
    del responses

    def _prep(ids):
        return (ids.T.reshape(_RT, 8, 32, 128)
                .transpose(0, 2, 1, 3).reshape(_RT * 32 * 8, 128))

    sid_sc = _prep(student_ids)
    qid_sc = _prep(questions)
    btT = _make_btab(beta_base, beta_gaps)
    qtab = jnp.concatenate([alpha_raw.T, btT], axis=0).T
    th128 = jnp.pad(theta_table, ((0, 7), (0, 0))).reshape(125001, 128)
    th128 = jax.lax.optimization_barrier(th128)
    th_in = th128.reshape(1000008, 16)
    t5, a5, b4, l5, p5 = _sc_run(sid_sc, qid_sc, th_in, qtab)
    theta = jnp.transpose(t5, (2, 4, 0, 1, 3)).reshape(_B, _S, _D)
    alpha = jnp.transpose(a5, (2, 4, 0, 1, 3)).reshape(_B, _S, _D)
    beta = jnp.transpose(b4, (1, 3, 0, 2)).reshape(_B, _S, 4)
    logits = jnp.transpose(l5, (2, 4, 1, 3, 0)).reshape(_B, _S, _K)
    probs = jnp.transpose(p5, (2, 4, 1, 3, 0)).reshape(_B, _S, _K)
    return (theta, alpha, beta, logits, probs)


# speedup vs baseline: 1.3709x; 1.3709x over previous
import jax
import jax.numpy as jnp
from jax import lax
from jax.experimental import pallas as pl
from jax.experimental.pallas import tpu as pltpu
from jax.experimental.pallas import tpu_sc as plsc

_B, _S = 4096, 200
_N = _B * _S
_D = 16
_K = 5
_NC, _NS = 2, 16
_NW = _NC * _NS
_RT = _S // 8
_Q1 = 100001
_HU = 4
_HP = _HU * 128

_BBLK = 2048


def _btab_body(bb_ref, bg_ref, out_ref):
    b0 = bb_ref[...]
    g = jax.nn.softplus(bg_ref[...])
    b1 = b0 + g[0:1, :]
    b2 = b1 + g[1:2, :]
    b3 = b2 + g[2:3, :]
    z = jnp.zeros_like(b0)
    out_ref[...] = jnp.concatenate([b0, b1, b2, b3, z, z, z, z], axis=0)


def _make_btab(beta_base, beta_gaps):
    grid = (_Q1 + _BBLK - 1) // _BBLK
    btT = pl.pallas_call(
        _btab_body,
        grid=(grid,),
        in_specs=[
            pl.BlockSpec((1, _BBLK), lambda i: (0, i)),
            pl.BlockSpec((3, _BBLK), lambda i: (0, i)),
        ],
        out_specs=pl.BlockSpec((8, _BBLK), lambda i: (0, i)),
        out_shape=jax.ShapeDtypeStruct((8, _Q1), jnp.float32),
    )(beta_base.T, beta_gaps.T)
    return btT.T


def _sc_body(sid_hbm, qid_hbm, th_hbm, al_hbm, bt_hbm,
             t5, a5, b4, l5, p5,
             sidx0, sidx1, qidx0, qidx1,
             thv0, thv1, alv0, alv1, btv0, btv1,
             thT0, thT1, alT0, alT1, btT0, btT1,
             lgT0, lgT1, prT0, prT1,
             semg0, semg1, semo0, semo1):
    w = lax.axis_index("s") * _NC + lax.axis_index("c")
    lane = jnp.arange(16, dtype=jnp.int32)
    cols = [jnp.full((16,), d, jnp.int32) for d in range(16)]
    SIDX = (sidx0, sidx1)
    QIDX = (qidx0, qidx1)
    THV = (thv0, thv1)
    ALV = (alv0, alv1)
    BTV = (btv0, btv1)
    THT = (thT0, thT1)
    ALT = (alT0, alT1)
    BTT = (btT0, btT1)
    LGT = (lgT0, lgT1)
    PRT = (prT0, prT1)
    SEMG = (semg0, semg1)
    SEMO = (semo0, semo1)

    def fire(rt, h, par):
        ibase = (rt * _NW + w) * 8 + h * _HU
        pltpu.sync_copy(sid_hbm.at[pl.ds(ibase, _HU)], SIDX[par])
        pltpu.sync_copy(qid_hbm.at[pl.ds(ibase, _HU)], QIDX[par])
        for u in range(_HU):
            dst = pl.ds(u * 128, 128)
            pltpu.async_copy(th_hbm.at[SIDX[par].at[u]], THV[par].at[dst], SEMG[par])
            pltpu.async_copy(al_hbm.at[QIDX[par].at[u]], ALV[par].at[dst], SEMG[par])
            pltpu.async_copy(bt_hbm.at[QIDX[par].at[u]], BTV[par].at[dst], SEMG[par])

    def drain_gathers(par):
        pltpu.make_async_copy(th_hbm.at[pl.ds(0, _HP)], THV[par], SEMG[par]).wait()
        pltpu.make_async_copy(al_hbm.at[pl.ds(0, _HP)], ALV[par], SEMG[par]).wait()
        pltpu.make_async_copy(bt_hbm.at[pl.ds(0, _HP)], BTV[par], SEMG[par]).wait()

    def drain_out(par):
        for _i in range(4 * _HU):
            pltpu.make_async_copy(t5.at[0, 0, 0], THT[par].at[0, pl.ds(0, 8)],
                                  SEMO[par]).wait()
        for _i in range(_HU):
            pltpu.make_async_copy(b4.at[0, 0], BTT[par].at[0], SEMO[par]).wait()
        for _i in range(2 * _K):
            pltpu.make_async_copy(b4.at[0, 0], LGT[par].at[0], SEMO[par]).wait()

    def compute_half(rt, h, par):
        thv, alv, btv = THV[par], ALV[par], BTV[par]
        thT, alT, btT = THT[par], ALT[par], BTT[par]
        lgT, prT = LGT[par], PRT[par]
        sem_o = SEMO[par]

        @plsc.parallel_loop(0, _HU * 8, unroll=2)
        def _body(i):
            u = i // 8
            rows = i * 16 + lane
            sl = pl.ds((i % 8) * 16, 16)
            prods = []
            for d in range(16):
                tv = plsc.load_gather(thv, [rows, cols[d]])
                av = jnp.exp(0.3 * plsc.load_gather(alv, [rows, cols[d]]))
                thT[u, d, sl] = tv
                alT[u, d, sl] = av
                prods.append(av * tv)
            while len(prods) > 1:
                prods = [a + b for a, b in zip(prods[::2], prods[1::2])]
            at = prods[0]
            bv = []
            for k in range(4):
                b = plsc.load_gather(btv, [rows, cols[k]])
                btT[u, k, sl] = b
                bv.append(b)
            c1 = at - bv[0]
            c2 = c1 + at - bv[1]
            c3 = c2 + at - bv[2]
            c4 = c3 + at - bv[3]
            z = jnp.zeros((16,), jnp.float32)
            m = jnp.maximum(jnp.maximum(jnp.maximum(jnp.maximum(z, c1), c2), c3), c4)
            e0 = jnp.exp(z - m)
            e1 = jnp.exp(c1 - m)
            e2 = jnp.exp(c2 - m)
            e3 = jnp.exp(c3 - m)
            e4 = jnp.exp(c4 - m)
            r = 1.0 / (e0 + e1 + e2 + e3 + e4)
            lgT[0, u, sl] = z
            lgT[1, u, sl] = c1
            lgT[2, u, sl] = c2
            lgT[3, u, sl] = c3
            lgT[4, u, sl] = c4
            prT[0, u, sl] = e0 * r
            prT[1, u, sl] = e1 * r
            prT[2, u, sl] = e2 * r
            prT[3, u, sl] = e3 * r
            prT[4, u, sl] = e4 * r

        for u in range(_HU):
            s = rt * 8 + h * _HU + u
            pltpu.async_copy(thT.at[u, pl.ds(0, 8)], t5.at[s, 0, w], sem_o)
            pltpu.async_copy(thT.at[u, pl.ds(8, 8)], t5.at[s, 1, w], sem_o)
            pltpu.async_copy(alT.at[u, pl.ds(0, 8)], a5.at[s, 0, w], sem_o)
            pltpu.async_copy(alT.at[u, pl.ds(8, 8)], a5.at[s, 1, w], sem_o)
            pltpu.async_copy(btT.at[u], b4.at[s, w], sem_o)
        osl = pl.ds(h * _HU, _HU)
        for k in range(_K):
            pltpu.async_copy(lgT.at[k], l5.at[k, rt, w, osl], sem_o)
            pltpu.async_copy(prT.at[k], p5.at[k, rt, w, osl], sem_o)

    fire(0, 0, 0)

    def step(rt, _):
        fire(rt, 1, 1)
        drain_gathers(0)

        @pl.when(rt > 0)
        def _():
            drain_out(0)

        compute_half(rt, 0, 0)
        fire(jnp.minimum(rt + 1, _RT - 1), 0, 0)
        drain_gathers(1)

        @pl.when(rt > 0)
        def _():
            drain_out(1)

        compute_half(rt, 1, 1)
        return 0

    lax.fori_loop(0, _RT, step, 0)
    drain_gathers(0)
    drain_out(0)
    drain_out(1)


def _sc_run(sid_sc, qid_sc, theta_table, alpha_raw, btab):
    mesh = plsc.VectorSubcoreMesh(core_axis_name="c", subcore_axis_name="s")
    f32 = jnp.float32
    return pl.kernel(
        _sc_body,
        out_type=(
            jax.ShapeDtypeStruct((_S, 2, 32, 8, 128), f32),
            jax.ShapeDtypeStruct((_S, 2, 32, 8, 128), f32),
            jax.ShapeDtypeStruct((_S, 32, 4, 128), f32),
            jax.ShapeDtypeStruct((_K, _RT, 32, 8, 128), f32),
            jax.ShapeDtypeStruct((_K, _RT, 32, 8, 128), f32),
        ),
        mesh=mesh,
        scratch_types=[
            pltpu.VMEM((_HU, 128), jnp.int32),
            pltpu.VMEM((_HU, 128), jnp.int32),
            pltpu.VMEM((_HU, 128), jnp.int32),
            pltpu.VMEM((_HU, 128), jnp.int32),
            pltpu.VMEM((_HP, 16), f32),
            pltpu.VMEM((_HP, 16), f32),
            pltpu.VMEM((_HP, 16), f32),
            pltpu.VMEM((_HP, 16), f32),
            pltpu.VMEM((_HP, 8), f32),
            pltpu.VMEM((_HP, 8), f32),
            pltpu.VMEM((_HU, 16, 128), f32),
            pltpu.VMEM((_HU, 16, 128), f32),
            pltpu.VMEM((_HU, 16, 128), f32),
            pltpu.VMEM((_HU, 16, 128), f32),
            pltpu.VMEM((_HU, 4, 128), f32),
            pltpu.VMEM((_HU, 4, 128), f32),
            pltpu.VMEM((_K, _HU, 128), f32),
            pltpu.VMEM((_K, _HU, 128), f32),
            pltpu.VMEM((_K, _HU, 128), f32),
            pltpu.VMEM((_K, _HU, 128), f32),
            pltpu.SemaphoreType.DMA,
            pltpu.SemaphoreType.DMA,
            pltpu.SemaphoreType.DMA,
            pltpu.SemaphoreType.DMA,
        ],
        compiler_params=pltpu.CompilerParams(
            use_tc_tiling_on_sc=False, needs_layout_passes=False),
    )(sid_sc, qid_sc, theta_table, alpha_raw, btab)


@jax.jit
def kernel(student_ids, questions, responses, theta_table, alpha_raw, beta_base, beta_gaps):
    del responses

    def _prep(ids):
        return (ids.T.reshape(_RT, 8, 32, 128)
                .transpose(0, 2, 1, 3).reshape(_RT * 32 * 8, 128))

    sid_sc = _prep(student_ids)
    qid_sc = _prep(questions)
    btab = _make_btab(beta_base, beta_gaps)
    t5, a5, b4, l5, p5 = _sc_run(sid_sc, qid_sc, theta_table, alpha_raw, btab)
    theta = jnp.transpose(t5, (2, 4, 0, 1, 3)).reshape(_B, _S, _D)
    alpha = jnp.transpose(a5, (2, 4, 0, 1, 3)).reshape(_B, _S, _D)
    beta = jnp.transpose(b4, (1, 3, 0, 2)).reshape(_B, _S, 4)
    logits = jnp.transpose(l5, (2, 4, 1, 3, 0)).reshape(_B, _S, _K)
    probs = jnp.transpose(p5, (2, 4, 1, 3, 0)).reshape(_B, _S, _K)
    return (theta, alpha, beta, logits, probs)
